# baseline (speedup 1.0000x reference)
import functools

import jax
import jax.numpy as jnp
from jax import lax
from jax.experimental import pallas as pl
from jax.experimental.pallas import tpu as pltpu
from jax.experimental.pallas import tpu_sc as plsc

NC = 2
NS = 16
NW = NC * NS
LANES = 16
CH = 128


def _fill_f32(ref, n, value):
    v = jnp.full((LANES,), value, jnp.float32)

    def body(i, _):
        ref[pl.ds(i * LANES, LANES)] = v
        return 0

    lax.fori_loop(0, n // LANES, body, 0)


def _fill_rows_f32(ref, rows, cols, value):
    v = jnp.full((LANES,), value, jnp.float32)

    def body(i, _):
        r = i // (cols // LANES)
        c = (i % (cols // LANES)) * LANES
        ref[r, pl.ds(c, LANES)] = v
        return 0

    lax.fori_loop(0, rows * (cols // LANES), body, 0)


G = 13
GA = 10


def _make_deg_kernel(e, npad):
    assert e % CH == 0
    nch = e // CH
    full, rem = nch // NW, nch % NW
    assert full % G == 0
    rpt = npad // NS
    mesh = plsc.VectorSubcoreMesh(
        core_axis_name="c", subcore_axis_name="s", num_cores=NC, num_subcores=NS
    )

    @functools.partial(
        pl.kernel,
        out_type=jax.ShapeDtypeStruct((NC * npad,), jnp.float32),
        mesh=mesh,
        scratch_types=[
            pltpu.VMEM((full, CH), jnp.int32),
            pltpu.VMEM((CH,), jnp.int32),
            pltpu.VMEM((CH,), jnp.float32),
            pltpu.VMEM((rpt,), jnp.float32),
            pltpu.VMEM_SHARED((npad,), jnp.float32),
            pltpu.SemaphoreType.DMA,
        ],
        compiler_params=pltpu.CompilerParams(use_tc_tiling_on_sc=False),
    )
    def deg_kernel(idx_hbm, out_hbm, idx2_v, idxx_v, ones_v, zrow_v, deg_sh, sem):
        c = lax.axis_index("c")
        s = lax.axis_index("s")
        wid = c * NS + s
        _fill_f32(ones_v, CH, 1.0)
        _fill_f32(zrow_v, rpt, 0.0)
        pltpu.sync_copy(zrow_v, deg_sh.at[pl.ds(s * rpt, rpt)])
        pltpu.sync_copy(idx_hbm.at[pl.ds(wid * full, full)], idx2_v)
        plsc.subcore_barrier()

        @pl.loop(0, full, step=G)
        def _(k0):
            descs = [
                pltpu.async_copy(ones_v, deg_sh.at[idx2_v.at[k0 + j]], sem, add=True)
                for j in range(G)
            ]
            for dsc in descs:
                dsc.wait()

        @pl.when(wid < rem)
        def _():
            pltpu.sync_copy(idx_hbm.at[NW * full + wid], idxx_v)
            pltpu.sync_copy(ones_v, deg_sh.at[idxx_v], add=True)

        plsc.subcore_barrier()
        pltpu.sync_copy(
            deg_sh.at[pl.ds(s * rpt, rpt)], out_hbm.at[pl.ds(c * npad + s * rpt, rpt)]
        )

    return deg_kernel


def _make_agg_kernel(e, n, npad, feat, with_deg):
    assert e % CH == 0
    nch = e // CH
    full, rem = nch // NW, nch % NW
    rpt = npad // NS
    assert rpt % CH == 0
    mesh = plsc.VectorSubcoreMesh(
        core_axis_name="c", subcore_axis_name="s", num_cores=NC, num_subcores=NS
    )

    out_type = [jax.ShapeDtypeStruct((NC, npad, feat), jnp.float32)]
    scratch = [
        pltpu.VMEM((full, CH), jnp.int32),
        pltpu.VMEM((full, CH), jnp.int32),
        pltpu.VMEM((CH,), jnp.int32),
        pltpu.VMEM((CH,), jnp.int32),
        pltpu.VMEM((2, GA, CH, feat), jnp.float32),
        pltpu.VMEM_SHARED((npad, feat), jnp.float32),
        pltpu.SemaphoreType.DMA,
        pltpu.SemaphoreType.DMA,
        pltpu.SemaphoreType.DMA,
    ]
    if with_deg:
        out_type.append(jax.ShapeDtypeStruct((NC * npad,), jnp.float32))
        scratch += [
            pltpu.VMEM((CH,), jnp.float32),
            pltpu.VMEM((rpt,), jnp.float32),
            pltpu.VMEM_SHARED((npad,), jnp.float32),
        ]

    @functools.partial(
        pl.kernel,
        out_type=out_type,
        mesh=mesh,
        scratch_types=scratch,
        compiler_params=pltpu.CompilerParams(use_tc_tiling_on_sc=False),
    )
    def agg_kernel(tbl_hbm, src_hbm, dst_hbm, *rest):
        if with_deg:
            (agg_hbm, deg_hbm, idx_s2, idx_d2, idxx_s, idxx_d, rows_v, agg_sh,
             gsem0, gsem1, ssem, ones_v, zrow_v, deg_sh) = rest
        else:
            (agg_hbm, idx_s2, idx_d2, idxx_s, idxx_d, rows_v, agg_sh,
             gsem0, gsem1, ssem) = rest
        c = lax.axis_index("c")
        s = lax.axis_index("s")
        wid = c * NS + s
        _fill_rows_f32(rows_v.at[0, 0], CH, feat, 0.0)
        for j in range(rpt // CH):
            pltpu.sync_copy(rows_v.at[0, 0], agg_sh.at[pl.ds(s * rpt + j * CH, CH)])
        pltpu.sync_copy(src_hbm.at[pl.ds(wid * full, full)], idx_s2)
        pltpu.sync_copy(dst_hbm.at[pl.ds(wid * full, full)], idx_d2)
        if with_deg:
            _fill_f32(ones_v, CH, 1.0)
            _fill_f32(zrow_v, rpt, 0.0)
            pltpu.sync_copy(zrow_v, deg_sh.at[pl.ds(s * rpt, rpt)])
        plsc.subcore_barrier()

        groups = [(k0, min(GA, full - k0)) for k0 in range(0, full, GA)]
        gsems = (gsem0, gsem1)

        def fire_gathers(g, buf):
            k0, sz = groups[g]
            return [
                pltpu.async_copy(
                    tbl_hbm.at[idx_s2.at[k0 + j]], rows_v.at[buf, j], gsems[buf]
                )
                for j in range(sz)
            ]

        def fire_scatters(g, buf):
            k0, sz = groups[g]
            sd = [
                pltpu.async_copy(
                    rows_v.at[buf, j], agg_sh.at[idx_d2.at[k0 + j]], ssem, add=True
                )
                for j in range(sz)
            ]
            if with_deg:
                sd += [
                    pltpu.async_copy(
                        ones_v, deg_sh.at[idx_d2.at[k0 + j]], ssem, add=True
                    )
                    for j in range(sz)
                ]
            return sd

        pend = {0: fire_gathers(0, 0)}
        for g in range(len(groups)):
            b = g % 2
            if g + 1 < len(groups):
                pend[g + 1] = fire_gathers(g + 1, 1 - b)
            for dsc in pend.pop(g):
                dsc.wait()
            for dsc in fire_scatters(g, b):
                dsc.wait()

        @pl.when(wid < rem)
        def _():
            pltpu.sync_copy(src_hbm.at[NW * full + wid], idxx_s)
            pltpu.sync_copy(dst_hbm.at[NW * full + wid], idxx_d)
            pltpu.sync_copy(tbl_hbm.at[idxx_s], rows_v.at[0, 0])
            pltpu.sync_copy(rows_v.at[0, 0], agg_sh.at[idxx_d], add=True)
            if with_deg:
                pltpu.sync_copy(ones_v, deg_sh.at[idxx_d], add=True)

        plsc.subcore_barrier()
        pltpu.sync_copy(
            agg_sh.at[pl.ds(s * rpt, rpt)], agg_hbm.at[c, pl.ds(s * rpt, rpt)]
        )
        if with_deg:
            pltpu.sync_copy(
                deg_sh.at[pl.ds(s * rpt, rpt)],
                deg_hbm.at[pl.ds(c * npad + s * rpt, rpt)],
            )

    return agg_kernel


def _proj_body(x_ref, ns_ref, w_ref, out_ref):
    out_ref[...] = jnp.dot(
        x_ref[...] * ns_ref[...], w_ref[...], preferred_element_type=jnp.float32
    )


def _combine_body(agg_ref, nd_ref, ns_ref, b_ref, out_ref):
    a = agg_ref[0] + agg_ref[1]
    out_ref[...] = jax.nn.relu(nd_ref[...] * a + b_ref[...]) * ns_ref[...]


def _heads_body(agg_ref, nd_ref, w2_ref, b2_ref, w3_ref, b3_ref, mu_ref, lv_ref):
    a = (agg_ref[0] + agg_ref[1]) * nd_ref[...]
    mu_ref[...] = (
        jnp.dot(a, w2_ref[...], preferred_element_type=jnp.float32) + b2_ref[...]
    )
    lv_ref[...] = (
        jnp.dot(a, w3_ref[...], preferred_element_type=jnp.float32) + b3_ref[...]
    )


def _decoder_body(mu_i_ref, mu_all_ref, out_ref):
    out_ref[...] = lax.dot_general(
        mu_i_ref[...].astype(jnp.bfloat16),
        mu_all_ref[...].astype(jnp.bfloat16),
        (((1,), (1,)), ((), ())),
        preferred_element_type=jnp.float32,
    )


def kernel(x, edge_index, W1, b1, W2, b2, W3, b3):
    n, d = x.shape
    e = edge_index.shape[1]
    h1 = W1.shape[1]
    h2 = W2.shape[1]
    npad = ((n + NS * 16 - 1) // (NS * 16)) * NS * 16

    src = edge_index[0].reshape(e // CH, CH)
    dst = edge_index[1].reshape(e // CH, CH)

    degp_out = _make_deg_kernel(e, npad)(src).reshape(NC, npad)
    deg_out = degp_out[0, :n] + degp_out[1, :n]
    norm_src = jnp.where(
        deg_out > 0, lax.rsqrt(jnp.maximum(deg_out, 1.0)), 0.0
    )[:, None]

    rb = 1000
    p1 = pl.pallas_call(
        _proj_body,
        grid=(n // rb,),
        in_specs=[
            pl.BlockSpec((rb, d), lambda i: (i, 0)),
            pl.BlockSpec((rb, 1), lambda i: (i, 0)),
            pl.BlockSpec((d, h1), lambda i: (0, 0)),
        ],
        out_specs=pl.BlockSpec((rb, h1), lambda i: (i, 0)),
        out_shape=jax.ShapeDtypeStruct((n, h1), jnp.float32),
    )(x, norm_src, W1)

    aggp1, degp_in = _make_agg_kernel(e, n, npad, h1, True)(p1, src, dst)
    degp_in = degp_in.reshape(NC, npad)
    deg_in = degp_in[0, :n] + degp_in[1, :n]
    norm_dst = jnp.where(
        deg_in > 0, lax.rsqrt(jnp.maximum(deg_in, 1.0)), 0.0
    )[:, None]

    h1s = pl.pallas_call(
        _combine_body,
        grid=(n // rb,),
        in_specs=[
            pl.BlockSpec((NC, rb, h1), lambda i: (0, i, 0)),
            pl.BlockSpec((rb, 1), lambda i: (i, 0)),
            pl.BlockSpec((rb, 1), lambda i: (i, 0)),
            pl.BlockSpec((1, h1), lambda i: (0, 0)),
        ],
        out_specs=pl.BlockSpec((rb, h1), lambda i: (i, 0)),
        out_shape=jax.ShapeDtypeStruct((n, h1), jnp.float32),
    )(aggp1, norm_dst, norm_src, b1.reshape(1, h1))

    (aggp2,) = _make_agg_kernel(e, n, npad, h1, False)(h1s, src, dst)

    mu, logvar = pl.pallas_call(
        _heads_body,
        grid=(n // rb,),
        in_specs=[
            pl.BlockSpec((NC, rb, h1), lambda i: (0, i, 0)),
            pl.BlockSpec((rb, 1), lambda i: (i, 0)),
            pl.BlockSpec((h1, h2), lambda i: (0, 0)),
            pl.BlockSpec((1, h2), lambda i: (0, 0)),
            pl.BlockSpec((h1, h2), lambda i: (0, 0)),
            pl.BlockSpec((1, h2), lambda i: (0, 0)),
        ],
        out_specs=[
            pl.BlockSpec((rb, h2), lambda i: (i, 0)),
            pl.BlockSpec((rb, h2), lambda i: (i, 0)),
        ],
        out_shape=[
            jax.ShapeDtypeStruct((n, h2), jnp.float32),
            jax.ShapeDtypeStruct((n, h2), jnp.float32),
        ],
    )(aggp2, norm_dst, W2, b2.reshape(1, h2), W3, b3.reshape(1, h2))

    db = 400
    adj = pl.pallas_call(
        _decoder_body,
        grid=(n // db,),
        in_specs=[
            pl.BlockSpec((db, h2), lambda i: (i, 0)),
            pl.BlockSpec((n, h2), lambda i: (0, 0)),
        ],
        out_specs=pl.BlockSpec((db, n), lambda i: (i, 0)),
        out_shape=jax.ShapeDtypeStruct((n, n), jnp.float32),
    )(mu, mu)

    return (adj, mu, logvar)


# speedup vs baseline: 1.0162x; 1.0162x over previous
import functools

import jax
import jax.numpy as jnp
from jax import lax
from jax.experimental import pallas as pl
from jax.experimental.pallas import tpu as pltpu
from jax.experimental.pallas import tpu_sc as plsc

NC = 2
NS = 16
NW = NC * NS
LANES = 16
CH = 128


def _fill_f32(ref, n, value):
    v = jnp.full((LANES,), value, jnp.float32)

    def body(i, _):
        ref[pl.ds(i * LANES, LANES)] = v
        return 0

    lax.fori_loop(0, n // LANES, body, 0)


def _fill_rows_f32(ref, rows, cols, value):
    v = jnp.full((LANES,), value, jnp.float32)

    def body(i, _):
        r = i // (cols // LANES)
        c = (i % (cols // LANES)) * LANES
        ref[r, pl.ds(c, LANES)] = v
        return 0

    lax.fori_loop(0, rows * (cols // LANES), body, 0)


G = 13
GA = 10


def _make_deg_kernel(e, npad):
    assert e % CH == 0
    nch = e // CH
    full, rem = nch // NW, nch % NW
    assert full % G == 0
    rpt = npad // NS
    mesh = plsc.VectorSubcoreMesh(
        core_axis_name="c", subcore_axis_name="s", num_cores=NC, num_subcores=NS
    )

    @functools.partial(
        pl.kernel,
        out_type=jax.ShapeDtypeStruct((NC * npad,), jnp.float32),
        mesh=mesh,
        scratch_types=[
            pltpu.VMEM((full, CH), jnp.int32),
            pltpu.VMEM((CH,), jnp.int32),
            pltpu.VMEM((CH,), jnp.float32),
            pltpu.VMEM((rpt,), jnp.float32),
            pltpu.VMEM_SHARED((npad,), jnp.float32),
            pltpu.SemaphoreType.DMA,
        ],
        compiler_params=pltpu.CompilerParams(use_tc_tiling_on_sc=False),
    )
    def deg_kernel(idx_hbm, out_hbm, idx2_v, idxx_v, ones_v, zrow_v, deg_sh, sem):
        c = lax.axis_index("c")
        s = lax.axis_index("s")
        wid = c * NS + s
        _fill_f32(ones_v, CH, 1.0)
        _fill_f32(zrow_v, rpt, 0.0)
        pltpu.sync_copy(zrow_v, deg_sh.at[pl.ds(s * rpt, rpt)])
        pltpu.sync_copy(idx_hbm.at[pl.ds(wid * full, full)], idx2_v)
        plsc.subcore_barrier()

        @pl.loop(0, full, step=G)
        def _(k0):
            descs = [
                pltpu.async_copy(ones_v, deg_sh.at[idx2_v.at[k0 + j]], sem, add=True)
                for j in range(G)
            ]
            for dsc in descs:
                dsc.wait()

        @pl.when(wid < rem)
        def _():
            pltpu.sync_copy(idx_hbm.at[NW * full + wid], idxx_v)
            pltpu.sync_copy(ones_v, deg_sh.at[idxx_v], add=True)

        plsc.subcore_barrier()
        pltpu.sync_copy(
            deg_sh.at[pl.ds(s * rpt, rpt)], out_hbm.at[pl.ds(c * npad + s * rpt, rpt)]
        )

    return deg_kernel


def _make_agg_kernel(e, n, npad, feat, with_deg):
    assert e % CH == 0
    nch = e // CH
    full, rem = nch // NW, nch % NW
    rpt = npad // NS
    assert rpt % CH == 0
    mesh = plsc.VectorSubcoreMesh(
        core_axis_name="c", subcore_axis_name="s", num_cores=NC, num_subcores=NS
    )

    out_type = [jax.ShapeDtypeStruct((NC, npad, feat), jnp.float32)]
    scratch = [
        pltpu.VMEM((full, CH), jnp.int32),
        pltpu.VMEM((full, CH), jnp.int32),
        pltpu.VMEM((CH,), jnp.int32),
        pltpu.VMEM((CH,), jnp.int32),
        pltpu.VMEM((2, GA, CH, feat), jnp.float32),
        pltpu.VMEM_SHARED((npad, feat), jnp.float32),
        pltpu.SemaphoreType.DMA,
        pltpu.SemaphoreType.DMA,
        pltpu.SemaphoreType.DMA,
    ]
    if with_deg:
        out_type.append(jax.ShapeDtypeStruct((NC * npad,), jnp.float32))
        scratch += [
            pltpu.VMEM((CH,), jnp.float32),
            pltpu.VMEM((rpt,), jnp.float32),
            pltpu.VMEM_SHARED((npad,), jnp.float32),
        ]

    @functools.partial(
        pl.kernel,
        out_type=out_type,
        mesh=mesh,
        scratch_types=scratch,
        compiler_params=pltpu.CompilerParams(use_tc_tiling_on_sc=False),
    )
    def agg_kernel(tbl_hbm, src_hbm, dst_hbm, *rest):
        if with_deg:
            (agg_hbm, deg_hbm, idx_s2, idx_d2, idxx_s, idxx_d, rows_v, agg_sh,
             gsem0, gsem1, ssem, ones_v, zrow_v, deg_sh) = rest
        else:
            (agg_hbm, idx_s2, idx_d2, idxx_s, idxx_d, rows_v, agg_sh,
             gsem0, gsem1, ssem) = rest
        c = lax.axis_index("c")
        s = lax.axis_index("s")
        wid = c * NS + s
        _fill_rows_f32(rows_v.at[0, 0], CH, feat, 0.0)
        setup = [
            pltpu.async_copy(
                rows_v.at[0, 0], agg_sh.at[pl.ds(s * rpt + j * CH, CH)], ssem
            )
            for j in range(rpt // CH)
        ]
        setup.append(
            pltpu.async_copy(src_hbm.at[pl.ds(wid * full, full)], idx_s2, gsem0)
        )
        setup.append(
            pltpu.async_copy(dst_hbm.at[pl.ds(wid * full, full)], idx_d2, gsem1)
        )
        if with_deg:
            _fill_f32(ones_v, CH, 1.0)
            _fill_f32(zrow_v, rpt, 0.0)
            setup.append(
                pltpu.async_copy(zrow_v, deg_sh.at[pl.ds(s * rpt, rpt)], ssem)
            )
        for dsc in setup:
            dsc.wait()
        plsc.subcore_barrier()

        groups = [(k0, min(GA, full - k0)) for k0 in range(0, full, GA)]
        gsems = (gsem0, gsem1)

        def fire_gathers(g, buf):
            k0, sz = groups[g]
            return [
                pltpu.async_copy(
                    tbl_hbm.at[idx_s2.at[k0 + j]], rows_v.at[buf, j], gsems[buf]
                )
                for j in range(sz)
            ]

        def fire_scatters(g, buf):
            k0, sz = groups[g]
            sd = [
                pltpu.async_copy(
                    rows_v.at[buf, j], agg_sh.at[idx_d2.at[k0 + j]], ssem, add=True
                )
                for j in range(sz)
            ]
            if with_deg:
                sd += [
                    pltpu.async_copy(
                        ones_v, deg_sh.at[idx_d2.at[k0 + j]], ssem, add=True
                    )
                    for j in range(sz)
                ]
            return sd

        pend = {0: fire_gathers(0, 0)}
        for g in range(len(groups)):
            b = g % 2
            if g + 1 < len(groups):
                pend[g + 1] = fire_gathers(g + 1, 1 - b)
            for dsc in pend.pop(g):
                dsc.wait()
            for dsc in fire_scatters(g, b):
                dsc.wait()

        @pl.when(wid < rem)
        def _():
            pltpu.sync_copy(src_hbm.at[NW * full + wid], idxx_s)
            pltpu.sync_copy(dst_hbm.at[NW * full + wid], idxx_d)
            pltpu.sync_copy(tbl_hbm.at[idxx_s], rows_v.at[0, 0])
            pltpu.sync_copy(rows_v.at[0, 0], agg_sh.at[idxx_d], add=True)
            if with_deg:
                pltpu.sync_copy(ones_v, deg_sh.at[idxx_d], add=True)

        plsc.subcore_barrier()
        outd = [
            pltpu.async_copy(
                agg_sh.at[pl.ds(s * rpt, rpt)], agg_hbm.at[c, pl.ds(s * rpt, rpt)],
                gsem0,
            )
        ]
        if with_deg:
            outd.append(
                pltpu.async_copy(
                    deg_sh.at[pl.ds(s * rpt, rpt)],
                    deg_hbm.at[pl.ds(c * npad + s * rpt, rpt)],
                    gsem1,
                )
            )
        for dsc in outd:
            dsc.wait()

    return agg_kernel


def _proj_body(x_ref, ns_ref, w_ref, out_ref):
    out_ref[...] = jnp.dot(
        x_ref[...] * ns_ref[...], w_ref[...], preferred_element_type=jnp.float32
    )


def _combine_body(agg_ref, nd_ref, ns_ref, b_ref, out_ref):
    a = agg_ref[0] + agg_ref[1]
    out_ref[...] = jax.nn.relu(nd_ref[...] * a + b_ref[...]) * ns_ref[...]


def _heads_body(agg_ref, nd_ref, w2_ref, b2_ref, w3_ref, b3_ref, mu_ref, lv_ref):
    a = (agg_ref[0] + agg_ref[1]) * nd_ref[...]
    mu_ref[...] = (
        jnp.dot(a, w2_ref[...], preferred_element_type=jnp.float32) + b2_ref[...]
    )
    lv_ref[...] = (
        jnp.dot(a, w3_ref[...], preferred_element_type=jnp.float32) + b3_ref[...]
    )


def _decoder_body(mu_i_ref, mu_all_ref, out_ref):
    out_ref[...] = lax.dot_general(
        mu_i_ref[...],
        mu_all_ref[...],
        (((1,), (1,)), ((), ())),
        preferred_element_type=jnp.float32,
    )


def kernel(x, edge_index, W1, b1, W2, b2, W3, b3):
    n, d = x.shape
    e = edge_index.shape[1]
    h1 = W1.shape[1]
    h2 = W2.shape[1]
    npad = ((n + NS * 16 - 1) // (NS * 16)) * NS * 16

    src = edge_index[0].reshape(e // CH, CH)
    dst = edge_index[1].reshape(e // CH, CH)

    degp_out = _make_deg_kernel(e, npad)(src).reshape(NC, npad)
    deg_out = degp_out[0, :n] + degp_out[1, :n]
    norm_src = jnp.where(
        deg_out > 0, lax.rsqrt(jnp.maximum(deg_out, 1.0)), 0.0
    )[:, None]

    rb = 1000
    p1 = pl.pallas_call(
        _proj_body,
        grid=(n // rb,),
        in_specs=[
            pl.BlockSpec((rb, d), lambda i: (i, 0)),
            pl.BlockSpec((rb, 1), lambda i: (i, 0)),
            pl.BlockSpec((d, h1), lambda i: (0, 0)),
        ],
        out_specs=pl.BlockSpec((rb, h1), lambda i: (i, 0)),
        out_shape=jax.ShapeDtypeStruct((n, h1), jnp.float32),
    )(x, norm_src, W1)

    aggp1, degp_in = _make_agg_kernel(e, n, npad, h1, True)(p1, src, dst)
    degp_in = degp_in.reshape(NC, npad)
    deg_in = degp_in[0, :n] + degp_in[1, :n]
    norm_dst = jnp.where(
        deg_in > 0, lax.rsqrt(jnp.maximum(deg_in, 1.0)), 0.0
    )[:, None]

    h1s = pl.pallas_call(
        _combine_body,
        grid=(n // rb,),
        in_specs=[
            pl.BlockSpec((NC, rb, h1), lambda i: (0, i, 0)),
            pl.BlockSpec((rb, 1), lambda i: (i, 0)),
            pl.BlockSpec((rb, 1), lambda i: (i, 0)),
            pl.BlockSpec((1, h1), lambda i: (0, 0)),
        ],
        out_specs=pl.BlockSpec((rb, h1), lambda i: (i, 0)),
        out_shape=jax.ShapeDtypeStruct((n, h1), jnp.float32),
    )(aggp1, norm_dst, norm_src, b1.reshape(1, h1))

    (aggp2,) = _make_agg_kernel(e, n, npad, h1, False)(h1s, src, dst)

    mu, logvar = pl.pallas_call(
        _heads_body,
        grid=(n // rb,),
        in_specs=[
            pl.BlockSpec((NC, rb, h1), lambda i: (0, i, 0)),
            pl.BlockSpec((rb, 1), lambda i: (i, 0)),
            pl.BlockSpec((h1, h2), lambda i: (0, 0)),
            pl.BlockSpec((1, h2), lambda i: (0, 0)),
            pl.BlockSpec((h1, h2), lambda i: (0, 0)),
            pl.BlockSpec((1, h2), lambda i: (0, 0)),
        ],
        out_specs=[
            pl.BlockSpec((rb, h2), lambda i: (i, 0)),
            pl.BlockSpec((rb, h2), lambda i: (i, 0)),
        ],
        out_shape=[
            jax.ShapeDtypeStruct((n, h2), jnp.float32),
            jax.ShapeDtypeStruct((n, h2), jnp.float32),
        ],
    )(aggp2, norm_dst, W2, b2.reshape(1, h2), W3, b3.reshape(1, h2))

    db = 400
    adj = pl.pallas_call(
        _decoder_body,
        grid=(n // db,),
        in_specs=[
            pl.BlockSpec((db, h2), lambda i: (i, 0)),
            pl.BlockSpec((n, h2), lambda i: (0, 0)),
        ],
        out_specs=pl.BlockSpec((db, n), lambda i: (i, 0)),
        out_shape=jax.ShapeDtypeStruct((n, n), jnp.float32),
    )(mu, mu)

    return (adj, mu, logvar)
